# Initial kernel scaffold; baseline (speedup 1.0000x reference)
import jax
import jax.numpy as jnp
from jax.experimental import pallas as pl


def kernel(sparse_ids, hist_ids, pos_ids, neg_ids, table_sparse, table_item, W_a1, b_a1, g_a1, be_a1, al_a1, W_a2, b_a2, W_u1, b_u1, g_u1, be_u1, al_u1, W_u2, b_u2, g_u2, be_u2, al_u2):
    raise NotImplementedError("write your pallas kernel here")


# speedup vs baseline: 2.1351x; 2.1351x over previous
import functools

import jax
import jax.numpy as jnp
from jax import lax
from jax.experimental import pallas as pl
from jax.experimental.pallas import tpu as pltpu
from jax.experimental.pallas import tpu_sc as plsc

_EPS = 1e-5
_CH = 128
_NW = 32


def _sc_gather(table_item, table_sp, ids_item, ids_sp, D):
    ni = ids_item.shape[0]
    ns = ids_sp.shape[0]
    nci = ni // _CH
    ncs = ns // _CH
    iters_i = -(-nci // _NW)
    iters_s = -(-ncs // _NW)
    mesh = plsc.VectorSubcoreMesh(core_axis_name="c", subcore_axis_name="s")

    @functools.partial(
        pl.kernel,
        out_type=(jax.ShapeDtypeStruct((ni, D), jnp.float32),
                  jax.ShapeDtypeStruct((ns, D), jnp.float32)),
        mesh=mesh,
        scratch_types=[
            pltpu.VMEM((_CH,), jnp.int32),
            pltpu.VMEM((_CH, D), jnp.float32),
            pltpu.SemaphoreType.DMA,
        ],
        compiler_params=pltpu.CompilerParams(use_tc_tiling_on_sc=False),
    )
    def gather(ti_hbm, ts_hbm, idi_hbm, ids_hbm, oi_hbm, os_hbm,
               idx_v, rows_v, sem):
        wid = lax.axis_index("s") * 2 + lax.axis_index("c")

        def run(tbl, ids, out, nchunks, iters):
            for it in range(iters):
                c = wid + _NW * it

                @pl.when(c < nchunks)
                def _():
                    base = c * _CH
                    pltpu.sync_copy(ids.at[pl.ds(base, _CH)], idx_v)
                    pltpu.async_copy(tbl.at[idx_v], rows_v, sem).wait()
                    pltpu.sync_copy(rows_v, out.at[pl.ds(base, _CH)])

        run(ti_hbm, idi_hbm, oi_hbm, nci, iters_i)
        run(ts_hbm, ids_hbm, os_hbm, ncs, iters_s)

    return gather(table_item, table_sp, ids_item, ids_sp)


def _bn_dice(x, g, be, al):
    m = jnp.mean(x, axis=0, keepdims=True)
    xc = x - m
    v = jnp.mean(xc * xc, axis=0, keepdims=True)
    rs = lax.rsqrt(v + _EPS)
    xn = xc * rs
    bn = g * xn + be
    v2 = g * g * v * (rs * rs)
    s2 = lax.rsqrt(v2 + _EPS)
    pgate = jax.nn.sigmoid(g * s2 * xn)
    return bn * (al + pgate * (1.0 - al))


def _tc_body(hist_ref, pos_ref, neg_ref, sp_ref,
             wa1_ref, ba1_ref, ga1_ref, bea1_ref, ala1_ref,
             wa2_ref, ba2_ref,
             wu1_ref, bu1_ref, gu1_ref, beu1_ref, alu1_ref,
             wu2_ref, bu2_ref, gu2_ref, beu2_ref, alu2_ref,
             y_ref,
             tgt_scr, t13_scr, w23_scr, w4_scr, stat_scr, bnc_scr,
             att_scr):
    p = pl.program_id(0)
    l = pl.program_id(1)
    L, B, D = hist_ref.shape
    NNEG = neg_ref.shape[1]
    SPD = sp_ref.shape[1]
    cnt = float(B * L)

    @pl.when((p == 0) & (l == 0))
    def _init():
        pos = pos_ref[...]
        n = jnp.sqrt(jnp.sum(pos * pos, axis=1, keepdims=True))
        tgt = pos / jnp.maximum(n, 1e-12)
        tgt_scr[...] = tgt
        w13 = wa1_ref[0:D, :] + wa1_ref[2 * D:3 * D, :]
        w23_scr[...] = wa1_ref[D:2 * D, :] - wa1_ref[2 * D:3 * D, :]
        w4_scr[...] = wa1_ref[3 * D:4 * D, :]
        t13_scr[...] = (jnp.dot(tgt, w13, preferred_element_type=jnp.float32)
                        + ba1_ref[...])
        stat_scr[...] = jnp.zeros_like(stat_scr)

    def _h(hl):
        tgt = tgt_scr[...]
        return (jnp.dot(hl, w23_scr[...], preferred_element_type=jnp.float32)
                + jnp.dot(tgt * hl, w4_scr[...],
                          preferred_element_type=jnp.float32)
                + t13_scr[...])

    @pl.when(p == 0)
    def _phase0():
        h = _h(hist_ref[l])
        stat_scr[0:1, :] += jnp.sum(h, axis=0, keepdims=True)
        stat_scr[1:2, :] += jnp.sum(h * h, axis=0, keepdims=True)

    @pl.when((p == 1) & (l == 0))
    def _stats():
        m = stat_scr[0:1, :] / cnt
        ex2 = stat_scr[1:2, :] / cnt
        v = ex2 - m * m
        rs = lax.rsqrt(v + _EPS)
        g = ga1_ref[...]
        v2 = g * g * v * (rs * rs)
        s2 = lax.rsqrt(v2 + _EPS)
        bnc_scr[0:1, :] = m
        bnc_scr[1:2, :] = rs
        bnc_scr[2:3, :] = g * s2
        att_scr[...] = jnp.zeros_like(att_scr)

    @pl.when(p == 1)
    def _phase1():
        hl = hist_ref[l]
        h = _h(hl)
        xn = (h - bnc_scr[0:1, :]) * bnc_scr[1:2, :]
        bn = ga1_ref[...] * xn + bea1_ref[...]
        pgate = jax.nn.sigmoid(bnc_scr[2:3, :] * xn)
        al = ala1_ref[...]
        dice = bn * (al + pgate * (1.0 - al))
        wl = (jnp.sum(dice * wa2_ref[...], axis=1, keepdims=True)
              + ba2_ref[0, 0])
        att_scr[...] += wl * hl

    @pl.when((p == 1) & (l == L - 1))
    def _tower():
        att = att_scr[...]
        u = (jnp.dot(sp_ref[...], wu1_ref[0:SPD, :],
                     preferred_element_type=jnp.float32)
             + jnp.dot(att, wu1_ref[SPD:SPD + D, :],
                       preferred_element_type=jnp.float32)
             + bu1_ref[...])
        u = _bn_dice(u, gu1_ref[...], beu1_ref[...], alu1_ref[...])
        u = (jnp.dot(u, wu2_ref[...], preferred_element_type=jnp.float32)
             + bu2_ref[...])
        u = _bn_dice(u, gu2_ref[...], beu2_ref[...], alu2_ref[...])
        n = jnp.sqrt(jnp.sum(u * u, axis=1, keepdims=True))
        user = u / jnp.maximum(n, 1e-12)
        y_ref[:, 0:1] = jnp.sum(user * tgt_scr[...], axis=1, keepdims=True)
        ne = neg_ref[...]
        nn = jnp.sqrt(jnp.sum(ne * ne, axis=2, keepdims=True))
        nen = ne / jnp.maximum(nn, 1e-12)
        y_ref[:, 1:1 + NNEG] = jnp.sum(user[:, None, :] * nen, axis=2)


def _tc_dense(hist3, posr, negr, spf, wa1, ba1, ga1, bea1, ala1, wa2, ba2,
              wu1, bu1, gu1, beu1, alu1, wu2, bu2, gu2, beu2, alu2,
              interpret=False):
    L, B, D = hist3.shape
    NNEG = negr.shape[1]
    full = lambda a: pl.BlockSpec(a.shape, lambda p, l: (0,) * a.ndim)
    args = (hist3, posr, negr, spf, wa1, ba1, ga1, bea1, ala1, wa2, ba2,
            wu1, bu1, gu1, beu1, alu1, wu2, bu2, gu2, beu2, alu2)
    NA = wa1.shape[1]
    return pl.pallas_call(
        _tc_body,
        grid=(2, L),
        in_specs=[full(a) for a in args],
        out_specs=pl.BlockSpec((B, 1 + NNEG), lambda p, l: (0, 0)),
        out_shape=jax.ShapeDtypeStruct((B, 1 + NNEG), jnp.float32),
        scratch_shapes=[
            pltpu.VMEM((B, D), jnp.float32),
            pltpu.VMEM((B, NA), jnp.float32),
            pltpu.VMEM((D, NA), jnp.float32),
            pltpu.VMEM((D, NA), jnp.float32),
            pltpu.VMEM((2, NA), jnp.float32),
            pltpu.VMEM((3, NA), jnp.float32),
            pltpu.VMEM((B, D), jnp.float32),
        ],
        compiler_params=pltpu.CompilerParams(
            vmem_limit_bytes=63 * 1024 * 1024),
        interpret=interpret,
    )(*args)


def kernel(sparse_ids, hist_ids, pos_ids, neg_ids, table_sparse, table_item,
           W_a1, b_a1, g_a1, be_a1, al_a1, W_a2, b_a2,
           W_u1, b_u1, g_u1, be_u1, al_u1,
           W_u2, b_u2, g_u2, be_u2, al_u2):
    B, NS = sparse_ids.shape
    L = hist_ids.shape[1]
    NNEG = neg_ids.shape[1]
    VS = table_sparse.shape[1]
    D = table_item.shape[1]

    ids_item = jnp.concatenate([
        hist_ids.astype(jnp.int32).T.reshape(-1),
        pos_ids.astype(jnp.int32).reshape(-1),
        neg_ids.astype(jnp.int32).reshape(-1),
    ])
    ids_sp = (sparse_ids.astype(jnp.int32)
              + (jnp.arange(NS, dtype=jnp.int32) * VS)[None, :]).reshape(-1)

    out_item, out_sp = _sc_gather(
        table_item, table_sparse.reshape(NS * VS, D), ids_item, ids_sp, D)

    hist3 = out_item[:B * L].reshape(L, B, D)
    posr = out_item[B * L:B * L + B]
    negr = out_item[B * L + B:].reshape(B, NNEG, D)
    spf = out_sp.reshape(B, NS * D)

    row = lambda a: a.reshape(1, -1)
    return _tc_dense(hist3, posr, negr, spf,
                     W_a1, row(b_a1), row(g_a1), row(be_a1), row(al_a1),
                     W_a2.reshape(1, -1), b_a2.reshape(1, 1),
                     W_u1, row(b_u1), row(g_u1), row(be_u1), row(al_u1),
                     W_u2, row(b_u2), row(g_u2), row(be_u2), row(al_u2))


# speedup vs baseline: 2.3793x; 1.1144x over previous
import functools

import jax
import jax.numpy as jnp
from jax import lax
from jax.experimental import pallas as pl
from jax.experimental.pallas import tpu as pltpu
from jax.experimental.pallas import tpu_sc as plsc

_EPS = 1e-5
_CH = 128
_NW = 32
_W = 128


def _sc_gather(table_item, table_sp, ids_item, ids_sp):
    ni = ids_item.shape[0]
    ns = ids_sp.shape[0]
    nci = ni // _CH
    ncs = ns // _CH
    iters_i = -(-nci // _NW)
    iters_s = -(-ncs // _NW)
    mesh = plsc.VectorSubcoreMesh(core_axis_name="c", subcore_axis_name="s")

    @functools.partial(
        pl.kernel,
        out_type=(jax.ShapeDtypeStruct((ni, _W), jnp.float32),
                  jax.ShapeDtypeStruct((ns, _W), jnp.float32)),
        mesh=mesh,
        scratch_types=[
            pltpu.VMEM((_CH,), jnp.int32),
            pltpu.VMEM((_CH, _W), jnp.float32),
            pltpu.SemaphoreType.DMA,
        ],
        compiler_params=pltpu.CompilerParams(use_tc_tiling_on_sc=False),
    )
    def gather(ti_hbm, ts_hbm, idi_hbm, ids_hbm, oi_hbm, os_hbm,
               idx_v, rows_v, sem):
        wid = lax.axis_index("s") * 2 + lax.axis_index("c")

        def run(tbl, ids, out, nchunks, iters):
            def body(i, carry):
                c = wid + _NW * i

                @pl.when(c < nchunks)
                def _():
                    base = c * _CH
                    pltpu.sync_copy(ids.at[pl.ds(base, _CH)], idx_v)
                    pltpu.async_copy(tbl.at[idx_v], rows_v, sem).wait()
                    pltpu.sync_copy(rows_v, out.at[pl.ds(base, _CH)])

                return carry

            lax.fori_loop(0, iters, body, 0)

        run(ti_hbm, idi_hbm, oi_hbm, nci, iters_i)
        run(ts_hbm, ids_hbm, os_hbm, ncs, iters_s)

    return gather(table_item, table_sp, ids_item, ids_sp)


def _bn_dice(x, g, be, al):
    m = jnp.mean(x, axis=0, keepdims=True)
    xc = x - m
    v = jnp.mean(xc * xc, axis=0, keepdims=True)
    rs = lax.rsqrt(v + _EPS)
    xn = xc * rs
    bn = g * xn + be
    v2 = g * g * v * (rs * rs)
    s2 = lax.rsqrt(v2 + _EPS)
    pgate = jax.nn.sigmoid(g * s2 * xn)
    return bn * (al + pgate * (1.0 - al))


def _l2n(x):
    n = jnp.sqrt(jnp.sum(x * x, axis=1, keepdims=True))
    return x / jnp.maximum(n, 1e-12)


def _tc_body(item_ref, sp_ref,
             wa1_ref, ba1_ref, ga1_ref, bea1_ref, ala1_ref,
             wa2_ref, ba2_ref,
             wu1a_ref, wu1b_ref, bu1_ref, gu1_ref, beu1_ref, alu1_ref,
             wu2_ref, bu2_ref, gu2_ref, beu2_ref, alu2_ref,
             y_ref,
             tgt_scr, t13_scr, w23_scr, w4_scr, stat_scr, bnc_scr, att_scr):
    p = pl.program_id(0)
    l = pl.program_id(1)
    NP, B, W = item_ref.shape
    L = pl.num_programs(1)
    NNEG = NP - L - 1
    D = W // 2
    cnt = float(B * L)

    @pl.when((p == 0) & (l == 0))
    def _init():
        tgt_scr[...] = _l2n(item_ref[L])
        zpad = jnp.zeros((W - D, wa1_ref.shape[1]), jnp.float32)
        w13 = jnp.concatenate(
            [wa1_ref[0:D, :] + wa1_ref[2 * D:3 * D, :], zpad], axis=0)
        w23_scr[0:D, :] = wa1_ref[D:2 * D, :] - wa1_ref[2 * D:3 * D, :]
        w23_scr[D:W, :] = zpad
        w4_scr[0:D, :] = wa1_ref[3 * D:4 * D, :]
        w4_scr[D:W, :] = zpad
        t13_scr[...] = (jnp.dot(tgt_scr[...], w13,
                                preferred_element_type=jnp.float32)
                        + ba1_ref[...])
        stat_scr[...] = jnp.zeros_like(stat_scr)

    def _h(hl):
        tgt = tgt_scr[...]
        return (jnp.dot(hl, w23_scr[...], preferred_element_type=jnp.float32)
                + jnp.dot(tgt * hl, w4_scr[...],
                          preferred_element_type=jnp.float32)
                + t13_scr[...])

    @pl.when(p == 0)
    def _phase0():
        h = _h(item_ref[l])
        stat_scr[0:1, :] += jnp.sum(h, axis=0, keepdims=True)
        stat_scr[1:2, :] += jnp.sum(h * h, axis=0, keepdims=True)

    @pl.when((p == 1) & (l == 0))
    def _stats():
        m = stat_scr[0:1, :] / cnt
        ex2 = stat_scr[1:2, :] / cnt
        v = ex2 - m * m
        rs = lax.rsqrt(v + _EPS)
        g = ga1_ref[...]
        v2 = g * g * v * (rs * rs)
        s2 = lax.rsqrt(v2 + _EPS)
        bnc_scr[0:1, :] = m
        bnc_scr[1:2, :] = rs
        bnc_scr[2:3, :] = g * s2
        att_scr[...] = jnp.zeros_like(att_scr)

    @pl.when(p == 1)
    def _phase1():
        hl = item_ref[l]
        h = _h(hl)
        xn = (h - bnc_scr[0:1, :]) * bnc_scr[1:2, :]
        bn = ga1_ref[...] * xn + bea1_ref[...]
        pgate = jax.nn.sigmoid(bnc_scr[2:3, :] * xn)
        al = ala1_ref[...]
        dice = bn * (al + pgate * (1.0 - al))
        wl = (jnp.sum(dice * wa2_ref[...], axis=1, keepdims=True)
              + ba2_ref[0, 0])
        att_scr[...] += wl * hl

    @pl.when((p == 1) & (l == L - 1))
    def _tower():
        u = (jnp.dot(sp_ref[...], wu1a_ref[...],
                     preferred_element_type=jnp.float32)
             + jnp.dot(att_scr[...], wu1b_ref[...],
                       preferred_element_type=jnp.float32)
             + bu1_ref[...])
        u = _bn_dice(u, gu1_ref[...], beu1_ref[...], alu1_ref[...])
        u = (jnp.dot(u, wu2_ref[...], preferred_element_type=jnp.float32)
             + bu2_ref[...])
        u = _bn_dice(u, gu2_ref[...], beu2_ref[...], alu2_ref[...])
        user = _l2n(u)
        y_ref[:, 0:1] = jnp.sum(user * tgt_scr[...], axis=1, keepdims=True)
        for j in range(NNEG):
            nen = _l2n(item_ref[L + 1 + j])
            y_ref[:, 1 + j:2 + j] = jnp.sum(user * nen, axis=1,
                                            keepdims=True)


def _tc_dense(item3, spf, wa1, ba1, ga1, bea1, ala1, wa2, ba2,
              wu1a, wu1b, bu1, gu1, beu1, alu1,
              wu2, bu2, gu2, beu2, alu2, interpret=False):
    NP, B, W = item3.shape
    L = 50
    NNEG = NP - L - 1
    NA = wa1.shape[1]
    full = lambda a: pl.BlockSpec(a.shape, lambda p, l: (0,) * a.ndim)
    args = (item3, spf, wa1, ba1, ga1, bea1, ala1, wa2, ba2,
            wu1a, wu1b, bu1, gu1, beu1, alu1, wu2, bu2, gu2, beu2, alu2)
    return pl.pallas_call(
        _tc_body,
        grid=(2, L),
        in_specs=[full(a) for a in args],
        out_specs=pl.BlockSpec((B, 1 + NNEG), lambda p, l: (0, 0)),
        out_shape=jax.ShapeDtypeStruct((B, 1 + NNEG), jnp.float32),
        scratch_shapes=[
            pltpu.VMEM((B, W), jnp.float32),
            pltpu.VMEM((B, NA), jnp.float32),
            pltpu.VMEM((W, NA), jnp.float32),
            pltpu.VMEM((W, NA), jnp.float32),
            pltpu.VMEM((2, NA), jnp.float32),
            pltpu.VMEM((3, NA), jnp.float32),
            pltpu.VMEM((B, W), jnp.float32),
        ],
        compiler_params=pltpu.CompilerParams(
            vmem_limit_bytes=63 * 1024 * 1024),
        interpret=interpret,
    )(*args)


def kernel(sparse_ids, hist_ids, pos_ids, neg_ids, table_sparse, table_item,
           W_a1, b_a1, g_a1, be_a1, al_a1, W_a2, b_a2,
           W_u1, b_u1, g_u1, be_u1, al_u1,
           W_u2, b_u2, g_u2, be_u2, al_u2):
    B, NS = sparse_ids.shape
    L = hist_ids.shape[1]
    NNEG = neg_ids.shape[1]
    VS = table_sparse.shape[1]
    D = table_item.shape[1]

    ti_pad = jnp.pad(table_item, ((0, 0), (0, _W - D)))
    ts_pad = jnp.pad(table_sparse.reshape(NS * VS, D), ((0, 0), (0, _W - D)))

    ids_item = jnp.concatenate([
        hist_ids.astype(jnp.int32).T.reshape(-1),
        pos_ids.astype(jnp.int32).reshape(-1),
        neg_ids.astype(jnp.int32).T.reshape(-1),
    ])
    ids_sp = (sparse_ids.astype(jnp.int32)
              + (jnp.arange(NS, dtype=jnp.int32) * VS)[None, :]).reshape(-1)

    out_item, out_sp = _sc_gather(ti_pad, ts_pad, ids_item, ids_sp)
    item3 = out_item.reshape(L + 1 + NNEG, B, _W)
    spf = out_sp.reshape(B, NS * _W)

    wu1a = jnp.pad(W_u1[:NS * D].reshape(NS, D, 256),
                   ((0, 0), (0, _W - D), (0, 0))).reshape(NS * _W, 256)
    wu1b = jnp.pad(W_u1[NS * D:], ((0, _W - D), (0, 0)))
    wu2 = jnp.pad(W_u2, ((0, 0), (0, _W - D)))
    row = lambda a: a.reshape(1, -1)
    padr = lambda a, v: jnp.pad(row(a), ((0, 0), (0, _W - D)),
                                constant_values=v)
    return _tc_dense(item3, spf,
                     W_a1, row(b_a1), row(g_a1), row(be_a1), row(al_a1),
                     W_a2.reshape(1, -1), b_a2.reshape(1, 1),
                     wu1a, wu1b, row(b_u1), row(g_u1), row(be_u1), row(al_u1),
                     wu2, padr(b_u2, 0.0), padr(g_u2, 1.0), padr(be_u2, 0.0),
                     padr(al_u2, 0.0))


# speedup vs baseline: 3.2961x; 1.3853x over previous
import functools

import jax
import jax.numpy as jnp
from jax import lax
from jax.experimental import pallas as pl
from jax.experimental.pallas import tpu as pltpu
from jax.experimental.pallas import tpu_sc as plsc

_EPS = 1e-5
_CH = 128
_NW = 32
_W = 128


def _relayout_pad(tview, vch):
    F, D, V = tview.shape
    nch = V // vch

    def body(in_ref, out_ref):
        x = in_ref[0]
        out_ref[0, :, 0:D] = x.T
        out_ref[0, :, D:_W] = jnp.zeros((vch, _W - D), jnp.float32)

    out = pl.pallas_call(
        body,
        grid=(F, nch),
        in_specs=[pl.BlockSpec((1, D, vch), lambda f, c: (f, 0, c))],
        out_specs=pl.BlockSpec((1, vch, _W), lambda f, c: (f * nch + c, 0, 0)),
        out_shape=jax.ShapeDtypeStruct((F * nch, vch, _W), jnp.float32),
    )(tview)
    return out.reshape(F * V, _W)


def _sc_gather(table, ids, n_out):
    nchunks = n_out // _CH
    iters = -(-nchunks // _NW)
    mesh = plsc.VectorSubcoreMesh(core_axis_name="c", subcore_axis_name="s")

    @functools.partial(
        pl.kernel,
        out_type=jax.ShapeDtypeStruct((n_out, _W), jnp.float32),
        mesh=mesh,
        scratch_types=[
            pltpu.VMEM((_CH,), jnp.int32),
            pltpu.VMEM((_CH, _W), jnp.float32),
            pltpu.SemaphoreType.DMA,
        ],
        compiler_params=pltpu.CompilerParams(use_tc_tiling_on_sc=False),
    )
    def gather(tbl, ids_hbm, out, idx_v, rows_v, sem):
        wid = lax.axis_index("s") * 2 + lax.axis_index("c")

        def body(i, carry):
            c = wid + _NW * i

            @pl.when(c < nchunks)
            def _():
                base = c * _CH
                pltpu.sync_copy(ids_hbm.at[pl.ds(base, _CH)], idx_v)
                pltpu.async_copy(tbl.at[idx_v], rows_v, sem).wait()
                pltpu.sync_copy(rows_v, out.at[pl.ds(base, _CH)])

            return carry

        lax.fori_loop(0, iters, body, 0)

    return gather(table, ids)


def _bn_dice(x, g, be, al):
    m = jnp.mean(x, axis=0, keepdims=True)
    xc = x - m
    v = jnp.mean(xc * xc, axis=0, keepdims=True)
    rs = lax.rsqrt(v + _EPS)
    xn = xc * rs
    bn = g * xn + be
    v2 = g * g * v * (rs * rs)
    s2 = lax.rsqrt(v2 + _EPS)
    pgate = jax.nn.sigmoid(g * s2 * xn)
    return bn * (al + pgate * (1.0 - al))


def _l2n(x):
    n = jnp.sqrt(jnp.sum(x * x, axis=1, keepdims=True))
    return x / jnp.maximum(n, 1e-12)


def _tc_body(item_ref, sp_ref,
             wa1_ref, ba1_ref, ga1_ref, bea1_ref, ala1_ref,
             wa2_ref, ba2_ref,
             wu1a_ref, wu1b_ref, bu1_ref, gu1_ref, beu1_ref, alu1_ref,
             wu2_ref, bu2_ref, gu2_ref, beu2_ref, alu2_ref,
             y_ref,
             tgt_scr, t13_scr, w23_scr, w4_scr, stat_scr, bnc_scr, att_scr):
    p = pl.program_id(0)
    l = pl.program_id(1)
    NP, B, W = item_ref.shape
    L = pl.num_programs(1)
    NNEG = NP - L - 1
    D = W // 2
    cnt = float(B * L)

    @pl.when((p == 0) & (l == 0))
    def _init():
        tgt_scr[...] = _l2n(item_ref[L])
        zpad = jnp.zeros((W - D, wa1_ref.shape[1]), jnp.float32)
        w13 = jnp.concatenate(
            [wa1_ref[0:D, :] + wa1_ref[2 * D:3 * D, :], zpad], axis=0)
        w23_scr[0:D, :] = wa1_ref[D:2 * D, :] - wa1_ref[2 * D:3 * D, :]
        w23_scr[D:W, :] = zpad
        w4_scr[0:D, :] = wa1_ref[3 * D:4 * D, :]
        w4_scr[D:W, :] = zpad
        t13_scr[...] = (jnp.dot(tgt_scr[...], w13,
                                preferred_element_type=jnp.float32)
                        + ba1_ref[...])
        stat_scr[...] = jnp.zeros_like(stat_scr)

    def _h(hl):
        tgt = tgt_scr[...]
        return (jnp.dot(hl, w23_scr[...], preferred_element_type=jnp.float32)
                + jnp.dot(tgt * hl, w4_scr[...],
                          preferred_element_type=jnp.float32)
                + t13_scr[...])

    @pl.when(p == 0)
    def _phase0():
        h = _h(item_ref[l])
        stat_scr[0:1, :] += jnp.sum(h, axis=0, keepdims=True)
        stat_scr[1:2, :] += jnp.sum(h * h, axis=0, keepdims=True)

    @pl.when((p == 1) & (l == 0))
    def _stats():
        m = stat_scr[0:1, :] / cnt
        ex2 = stat_scr[1:2, :] / cnt
        v = ex2 - m * m
        rs = lax.rsqrt(v + _EPS)
        g = ga1_ref[...]
        v2 = g * g * v * (rs * rs)
        s2 = lax.rsqrt(v2 + _EPS)
        bnc_scr[0:1, :] = m
        bnc_scr[1:2, :] = rs
        bnc_scr[2:3, :] = g * s2
        att_scr[...] = jnp.zeros_like(att_scr)

    @pl.when(p == 1)
    def _phase1():
        hl = item_ref[l]
        h = _h(hl)
        xn = (h - bnc_scr[0:1, :]) * bnc_scr[1:2, :]
        bn = ga1_ref[...] * xn + bea1_ref[...]
        pgate = jax.nn.sigmoid(bnc_scr[2:3, :] * xn)
        al = ala1_ref[...]
        dice = bn * (al + pgate * (1.0 - al))
        wl = (jnp.sum(dice * wa2_ref[...], axis=1, keepdims=True)
              + ba2_ref[0, 0])
        att_scr[...] += wl * hl

    @pl.when((p == 1) & (l == L - 1))
    def _tower():
        u = (jnp.dot(sp_ref[...], wu1a_ref[...],
                     preferred_element_type=jnp.float32)
             + jnp.dot(att_scr[...], wu1b_ref[...],
                       preferred_element_type=jnp.float32)
             + bu1_ref[...])
        u = _bn_dice(u, gu1_ref[...], beu1_ref[...], alu1_ref[...])
        u = (jnp.dot(u, wu2_ref[...], preferred_element_type=jnp.float32)
             + bu2_ref[...])
        u = _bn_dice(u, gu2_ref[...], beu2_ref[...], alu2_ref[...])
        user = _l2n(u)
        y_ref[:, 0:1] = jnp.sum(user * tgt_scr[...], axis=1, keepdims=True)
        for j in range(NNEG):
            nen = _l2n(item_ref[L + 1 + j])
            y_ref[:, 1 + j:2 + j] = jnp.sum(user * nen, axis=1,
                                            keepdims=True)


def _tc_dense(item3, spf, wa1, ba1, ga1, bea1, ala1, wa2, ba2,
              wu1a, wu1b, bu1, gu1, beu1, alu1,
              wu2, bu2, gu2, beu2, alu2, interpret=False):
    NP, B, W = item3.shape
    L = 50
    NNEG = NP - L - 1
    NA = wa1.shape[1]
    full = lambda a: pl.BlockSpec(a.shape, lambda p, l: (0,) * a.ndim)
    args = (item3, spf, wa1, ba1, ga1, bea1, ala1, wa2, ba2,
            wu1a, wu1b, bu1, gu1, beu1, alu1, wu2, bu2, gu2, beu2, alu2)
    return pl.pallas_call(
        _tc_body,
        grid=(2, L),
        in_specs=[full(a) for a in args],
        out_specs=pl.BlockSpec((B, 1 + NNEG), lambda p, l: (0, 0)),
        out_shape=jax.ShapeDtypeStruct((B, 1 + NNEG), jnp.float32),
        scratch_shapes=[
            pltpu.VMEM((B, W), jnp.float32),
            pltpu.VMEM((B, NA), jnp.float32),
            pltpu.VMEM((W, NA), jnp.float32),
            pltpu.VMEM((W, NA), jnp.float32),
            pltpu.VMEM((2, NA), jnp.float32),
            pltpu.VMEM((3, NA), jnp.float32),
            pltpu.VMEM((B, W), jnp.float32),
        ],
        compiler_params=pltpu.CompilerParams(
            vmem_limit_bytes=63 * 1024 * 1024),
        interpret=interpret,
    )(*args)


def kernel(sparse_ids, hist_ids, pos_ids, neg_ids, table_sparse, table_item,
           W_a1, b_a1, g_a1, be_a1, al_a1, W_a2, b_a2,
           W_u1, b_u1, g_u1, be_u1, al_u1,
           W_u2, b_u2, g_u2, be_u2, al_u2):
    B, NS = sparse_ids.shape
    L = hist_ids.shape[1]
    NNEG = neg_ids.shape[1]
    VS = table_sparse.shape[1]
    D = table_item.shape[1]

    ti_view = jnp.pad(table_item.T, ((0, 0), (0, 102400 - table_item.shape[0])))
    ti_pad = _relayout_pad(ti_view[None], 12800)
    ts_pad = _relayout_pad(jnp.transpose(table_sparse, (0, 2, 1)), VS)

    ids_item = jnp.concatenate([
        hist_ids.astype(jnp.int32).T.reshape(-1),
        pos_ids.astype(jnp.int32).reshape(-1),
        neg_ids.astype(jnp.int32).T.reshape(-1),
    ])
    ids_sp = (sparse_ids.astype(jnp.int32)
              + (jnp.arange(NS, dtype=jnp.int32) * VS)[None, :]).reshape(-1)

    out_item = _sc_gather(ti_pad, ids_item, ids_item.shape[0])
    out_sp = _sc_gather(ts_pad, ids_sp, ids_sp.shape[0])
    item3 = out_item.reshape(L + 1 + NNEG, B, _W)
    spf = out_sp.reshape(B, NS * _W)

    wu1a = jnp.pad(W_u1[:NS * D].reshape(NS, D, 256),
                   ((0, 0), (0, _W - D), (0, 0))).reshape(NS * _W, 256)
    wu1b = jnp.pad(W_u1[NS * D:], ((0, _W - D), (0, 0)))
    wu2 = jnp.pad(W_u2, ((0, 0), (0, _W - D)))
    row = lambda a: a.reshape(1, -1)
    padr = lambda a, v: jnp.pad(row(a), ((0, 0), (0, _W - D)),
                                constant_values=v)
    return _tc_dense(item3, spf,
                     W_a1, row(b_a1), row(g_a1), row(be_a1), row(al_a1),
                     W_a2.reshape(1, -1), b_a2.reshape(1, 1),
                     wu1a, wu1b, row(b_u1), row(g_u1), row(be_u1), row(al_u1),
                     wu2, padr(b_u2, 0.0), padr(g_u2, 1.0), padr(be_u2, 0.0),
                     padr(al_u2, 0.0))


# speedup vs baseline: 3.8495x; 1.1679x over previous
import functools

import jax
import jax.numpy as jnp
from jax import lax
from jax.experimental import pallas as pl
from jax.experimental.pallas import tpu as pltpu
from jax.experimental.pallas import tpu_sc as plsc

_EPS = 1e-5
_CH = 128
_NW = 32
_W = 128


def _relayout_pad(tview, vch, nch):
    F, D, V = tview.shape

    def body(in_ref, out_ref):
        x = in_ref[0]
        out_ref[0, :, 0:D] = x.T
        out_ref[0, :, D:_W] = jnp.zeros((vch, _W - D), jnp.float32)

    out = pl.pallas_call(
        body,
        grid=(F, nch),
        in_specs=[pl.BlockSpec((1, D, vch), lambda f, c: (f, 0, c))],
        out_specs=pl.BlockSpec((1, vch, _W), lambda f, c: (f * nch + c, 0, 0)),
        out_shape=jax.ShapeDtypeStruct((F * nch, vch, _W), jnp.float32),
    )(tview)
    return out.reshape(F * nch * vch, _W)


def _sc_gather_item(table, ids, n_hist, n_pn):
    nch_h = n_hist // _CH
    nch_all = (n_hist + n_pn) // _CH
    iters = -(-nch_all // _NW)
    mesh = plsc.VectorSubcoreMesh(core_axis_name="c", subcore_axis_name="s")

    @functools.partial(
        pl.kernel,
        out_type=(jax.ShapeDtypeStruct((n_hist, _W), jnp.float32),
                  jax.ShapeDtypeStruct((n_pn, _W), jnp.float32)),
        mesh=mesh,
        scratch_types=[
            pltpu.VMEM((_CH,), jnp.int32),
            pltpu.VMEM((_CH, _W), jnp.float32),
            pltpu.SemaphoreType.DMA,
        ],
        compiler_params=pltpu.CompilerParams(use_tc_tiling_on_sc=False),
    )
    def gather(tbl, ids_hbm, out_h, out_pn, idx_v, rows_v, sem):
        wid = lax.axis_index("s") * 2 + lax.axis_index("c")

        def body(i, carry):
            c = wid + _NW * i

            @pl.when(c < nch_all)
            def _():
                pltpu.sync_copy(ids_hbm.at[pl.ds(c * _CH, _CH)], idx_v)
                pltpu.async_copy(tbl.at[idx_v], rows_v, sem).wait()

                @pl.when(c < nch_h)
                def _():
                    pltpu.sync_copy(rows_v, out_h.at[pl.ds(c * _CH, _CH)])

                @pl.when(c >= nch_h)
                def _():
                    pltpu.sync_copy(
                        rows_v, out_pn.at[pl.ds((c - nch_h) * _CH, _CH)])

            return carry

        lax.fori_loop(0, iters, body, 0)

    return gather(table, ids)


def _sc_gather(table, ids, n_out):
    nchunks = n_out // _CH
    iters = -(-nchunks // _NW)
    mesh = plsc.VectorSubcoreMesh(core_axis_name="c", subcore_axis_name="s")

    @functools.partial(
        pl.kernel,
        out_type=jax.ShapeDtypeStruct((n_out, _W), jnp.float32),
        mesh=mesh,
        scratch_types=[
            pltpu.VMEM((_CH,), jnp.int32),
            pltpu.VMEM((_CH, _W), jnp.float32),
            pltpu.SemaphoreType.DMA,
        ],
        compiler_params=pltpu.CompilerParams(use_tc_tiling_on_sc=False),
    )
    def gather(tbl, ids_hbm, out, idx_v, rows_v, sem):
        wid = lax.axis_index("s") * 2 + lax.axis_index("c")

        def body(i, carry):
            c = wid + _NW * i

            @pl.when(c < nchunks)
            def _():
                base = c * _CH
                pltpu.sync_copy(ids_hbm.at[pl.ds(base, _CH)], idx_v)
                pltpu.async_copy(tbl.at[idx_v], rows_v, sem).wait()
                pltpu.sync_copy(rows_v, out.at[pl.ds(base, _CH)])

            return carry

        lax.fori_loop(0, iters, body, 0)

    return gather(table, ids)


def _bn_dice(x, g, be, al):
    m = jnp.mean(x, axis=0, keepdims=True)
    xc = x - m
    v = jnp.mean(xc * xc, axis=0, keepdims=True)
    rs = lax.rsqrt(v + _EPS)
    xn = xc * rs
    bn = g * xn + be
    v2 = g * g * v * (rs * rs)
    s2 = lax.rsqrt(v2 + _EPS)
    pgate = jax.nn.sigmoid(g * s2 * xn)
    return bn * (al + pgate * (1.0 - al))


def _l2n(x):
    n = jnp.sqrt(jnp.sum(x * x, axis=1, keepdims=True))
    return x / jnp.maximum(n, 1e-12)


def _att_body(hist_ref, pn_ref,
              wa1_ref, ba1_ref, ga1_ref, bea1_ref, ala1_ref,
              wa2_ref, ba2_ref,
              att_ref,
              tgt_scr, t13_scr, w23_scr, w4_scr, stat_scr, bnc_scr):
    p = pl.program_id(0)
    l = pl.program_id(1)
    L, B, W = hist_ref.shape
    D = W // 2
    cnt = float(B * L)

    @pl.when((p == 0) & (l == 0))
    def _init():
        tgt_scr[...] = _l2n(pn_ref[0])
        zpad = jnp.zeros((W - D, wa1_ref.shape[1]), jnp.float32)
        w13 = jnp.concatenate(
            [wa1_ref[0:D, :] + wa1_ref[2 * D:3 * D, :], zpad], axis=0)
        w23_scr[0:D, :] = wa1_ref[D:2 * D, :] - wa1_ref[2 * D:3 * D, :]
        w23_scr[D:W, :] = zpad
        w4_scr[0:D, :] = wa1_ref[3 * D:4 * D, :]
        w4_scr[D:W, :] = zpad
        t13_scr[...] = (jnp.dot(tgt_scr[...], w13,
                                preferred_element_type=jnp.float32)
                        + ba1_ref[...])
        stat_scr[...] = jnp.zeros_like(stat_scr)

    def _h(hl):
        tgt = tgt_scr[...]
        return (jnp.dot(hl, w23_scr[...], preferred_element_type=jnp.float32)
                + jnp.dot(tgt * hl, w4_scr[...],
                          preferred_element_type=jnp.float32)
                + t13_scr[...])

    @pl.when(p == 0)
    def _phase0():
        h = _h(hist_ref[l])
        stat_scr[0:1, :] += jnp.sum(h, axis=0, keepdims=True)
        stat_scr[1:2, :] += jnp.sum(h * h, axis=0, keepdims=True)

    @pl.when((p == 1) & (l == 0))
    def _stats():
        m = stat_scr[0:1, :] / cnt
        ex2 = stat_scr[1:2, :] / cnt
        v = ex2 - m * m
        rs = lax.rsqrt(v + _EPS)
        g = ga1_ref[...]
        v2 = g * g * v * (rs * rs)
        s2 = lax.rsqrt(v2 + _EPS)
        bnc_scr[0:1, :] = m
        bnc_scr[1:2, :] = rs
        bnc_scr[2:3, :] = g * s2
        att_ref[...] = jnp.zeros_like(att_ref)

    @pl.when(p == 1)
    def _phase1():
        hl = hist_ref[l]
        h = _h(hl)
        xn = (h - bnc_scr[0:1, :]) * bnc_scr[1:2, :]
        bn = ga1_ref[...] * xn + bea1_ref[...]
        pgate = jax.nn.sigmoid(bnc_scr[2:3, :] * xn)
        al = ala1_ref[...]
        dice = bn * (al + pgate * (1.0 - al))
        wl = (jnp.sum(dice * wa2_ref[...], axis=1, keepdims=True)
              + ba2_ref[0, 0])
        att_ref[...] += wl * hl


def _attention(hist3, pn3, wa1, ba1, ga1, bea1, ala1, wa2, ba2):
    L, B, W = hist3.shape
    NA = wa1.shape[1]
    full = lambda a: pl.BlockSpec(a.shape, lambda p, l: (0,) * a.ndim)
    args = (hist3, pn3, wa1, ba1, ga1, bea1, ala1, wa2, ba2)
    return pl.pallas_call(
        _att_body,
        grid=(2, L),
        in_specs=[full(a) for a in args],
        out_specs=pl.BlockSpec((B, W), lambda p, l: (0, 0)),
        out_shape=jax.ShapeDtypeStruct((B, W), jnp.float32),
        scratch_shapes=[
            pltpu.VMEM((B, W), jnp.float32),
            pltpu.VMEM((B, NA), jnp.float32),
            pltpu.VMEM((W, NA), jnp.float32),
            pltpu.VMEM((W, NA), jnp.float32),
            pltpu.VMEM((2, NA), jnp.float32),
            pltpu.VMEM((3, NA), jnp.float32),
        ],
        compiler_params=pltpu.CompilerParams(
            vmem_limit_bytes=63 * 1024 * 1024),
    )(*args)


def _tower_body(sp_ref, pn_ref, att_ref, wu1a_ref, wu1b_ref,
                bu1_ref, gu1_ref, beu1_ref, alu1_ref,
                wu2_ref, bu2_ref, gu2_ref, beu2_ref, alu2_ref,
                y_ref):
    NS = sp_ref.shape[0]
    NNEG = pn_ref.shape[0] - 1
    u = (jnp.dot(att_ref[...], wu1b_ref[...],
                 preferred_element_type=jnp.float32) + bu1_ref[...])
    for f in range(NS):
        u += jnp.dot(sp_ref[f], wu1a_ref[f],
                     preferred_element_type=jnp.float32)
    u = _bn_dice(u, gu1_ref[...], beu1_ref[...], alu1_ref[...])
    u = (jnp.dot(u, wu2_ref[...], preferred_element_type=jnp.float32)
         + bu2_ref[...])
    u = _bn_dice(u, gu2_ref[...], beu2_ref[...], alu2_ref[...])
    user = _l2n(u)
    tgt = _l2n(pn_ref[0])
    y_ref[:, 0:1] = jnp.sum(user * tgt, axis=1, keepdims=True)
    for j in range(NNEG):
        nen = _l2n(pn_ref[1 + j])
        y_ref[:, 1 + j:2 + j] = jnp.sum(user * nen, axis=1, keepdims=True)


def _tower(sp3, pn3, att, wu1a, wu1b, bu1, gu1, beu1, alu1,
           wu2, bu2, gu2, beu2, alu2):
    B = att.shape[0]
    NNEG = pn3.shape[0] - 1
    full = lambda a: pl.BlockSpec(a.shape, lambda: (0,) * a.ndim)
    args = (sp3, pn3, att, wu1a, wu1b, bu1, gu1, beu1, alu1,
            wu2, bu2, gu2, beu2, alu2)
    return pl.pallas_call(
        _tower_body,
        in_specs=[full(a) for a in args],
        out_specs=pl.BlockSpec((B, 1 + NNEG), lambda: (0, 0)),
        out_shape=jax.ShapeDtypeStruct((B, 1 + NNEG), jnp.float32),
        compiler_params=pltpu.CompilerParams(
            vmem_limit_bytes=63 * 1024 * 1024),
    )(*args)


def kernel(sparse_ids, hist_ids, pos_ids, neg_ids, table_sparse, table_item,
           W_a1, b_a1, g_a1, be_a1, al_a1, W_a2, b_a2,
           W_u1, b_u1, g_u1, be_u1, al_u1,
           W_u2, b_u2, g_u2, be_u2, al_u2):
    B, NS = sparse_ids.shape
    L = hist_ids.shape[1]
    NNEG = neg_ids.shape[1]
    VS = table_sparse.shape[1]
    D = table_item.shape[1]

    ti_pad = _relayout_pad(table_item.T[None], 6400, 16)
    ts_view = jnp.transpose(table_sparse, (0, 2, 1))
    ts_view = lax.optimization_barrier((ts_view, ti_pad))[0]
    ts_pad = _relayout_pad(ts_view, VS, 1)

    ids_item = jnp.concatenate([
        hist_ids.astype(jnp.int32).T.reshape(-1),
        pos_ids.astype(jnp.int32).reshape(-1),
        neg_ids.astype(jnp.int32).T.reshape(-1),
    ])
    ids_sp = (sparse_ids.astype(jnp.int32).T
              + (jnp.arange(NS, dtype=jnp.int32) * VS)[:, None]).reshape(-1)

    out_hist, out_pn = _sc_gather_item(ti_pad, ids_item, B * L, B * (1 + NNEG))
    out_sp = _sc_gather(ts_pad, ids_sp, B * NS)
    hist3 = out_hist.reshape(L, B, _W)
    pn3 = out_pn.reshape(1 + NNEG, B, _W)
    sp3 = out_sp.reshape(NS, B, _W)

    wu1a = jnp.pad(W_u1[:NS * D].reshape(NS, D, 256),
                   ((0, 0), (0, _W - D), (0, 0)))
    wu1b = jnp.pad(W_u1[NS * D:], ((0, _W - D), (0, 0)))
    wu2 = jnp.pad(W_u2, ((0, 0), (0, _W - D)))
    row = lambda a: a.reshape(1, -1)
    padr = lambda a, v: jnp.pad(row(a), ((0, 0), (0, _W - D)),
                                constant_values=v)

    att = _attention(hist3, pn3, W_a1, row(b_a1), row(g_a1), row(be_a1),
                     row(al_a1), W_a2.reshape(1, -1), b_a2.reshape(1, 1))
    return _tower(sp3, pn3, att, wu1a, wu1b,
                  row(b_u1), row(g_u1), row(be_u1), row(al_u1),
                  wu2, padr(b_u2, 0.0), padr(g_u2, 1.0), padr(be_u2, 0.0),
                  padr(al_u2, 0.0))
